# baseline (speedup 1.0000x reference)
import functools

import jax
import jax.numpy as jnp
from jax import lax
from jax.experimental import pallas as pl
from jax.experimental.pallas import tpu as pltpu
from jax.experimental.pallas import tpu_sc as plsc

_N, _NF, _HID, _NCLS = 4096, 512, 256, 64
_TAU = 0.5
_NLAYER = 2
_F32 = jnp.float32


def _fc_body(x_ref, w_ref, b_ref, h_ref, u_ref):
    h = lax.dot_general(x_ref[...], w_ref[...], (((1,), (1,)), ((), ())),
                        preferred_element_type=_F32)
    h = h + b_ref[...]
    h_ref[...] = h
    n = jnp.sqrt(jnp.sum(h * h, axis=1, keepdims=True))
    u_ref[...] = h / jnp.maximum(n, 1e-12)


def _fc(x, W, b):
    BM = 512
    return pl.pallas_call(
        _fc_body,
        grid=(_N // BM,),
        in_specs=[pl.BlockSpec((BM, _NF), lambda i: (i, 0)),
                  pl.BlockSpec((_HID, _NF), lambda i: (0, 0)),
                  pl.BlockSpec((1, _HID), lambda i: (0, 0))],
        out_specs=[pl.BlockSpec((BM, _HID), lambda i: (i, 0)),
                   pl.BlockSpec((BM, _HID), lambda i: (i, 0))],
        out_shape=[jax.ShapeDtypeStruct((_N, _HID), _F32)] * 2,
    )(x, W, b.reshape(1, _HID))


def _spmv_norm_body(a_ref, z_ref, p_ref, v_ref):
    p = jnp.dot(a_ref[...], z_ref[...], preferred_element_type=_F32)
    p_ref[...] = p
    n = jnp.sqrt(jnp.sum(p * p, axis=1, keepdims=True))
    v_ref[...] = p / jnp.maximum(n, 1e-12)


def _spmv_norm(A, Z):
    BM = 256
    return pl.pallas_call(
        _spmv_norm_body,
        grid=(_N // BM,),
        in_specs=[pl.BlockSpec((BM, _N), lambda i: (i, 0)),
                  pl.BlockSpec((_N, _HID), lambda i: (0, 0))],
        out_specs=[pl.BlockSpec((BM, _HID), lambda i: (i, 0)),
                   pl.BlockSpec((BM, _HID), lambda i: (i, 0))],
        out_shape=[jax.ShapeDtypeStruct((_N, _HID), _F32)] * 2,
    )(A, Z)


def _spmv_body(a_ref, z_ref, p_ref):
    p_ref[...] = jnp.dot(a_ref[...], z_ref[...], preferred_element_type=_F32)


def _spmv(A, Z):
    BM = 256
    return pl.pallas_call(
        _spmv_body,
        grid=(_N // BM,),
        in_specs=[pl.BlockSpec((BM, _N), lambda i: (i, 0)),
                  pl.BlockSpec((_N, _HID), lambda i: (0, 0))],
        out_specs=pl.BlockSpec((BM, _HID), lambda i: (i, 0)),
        out_shape=jax.ShapeDtypeStruct((_N, _HID), _F32),
    )(A, Z)


def _sim_body(u_i, v1_i, v1_j, v2_j, a_ref, ct_ref, r1_acc, r2_acc, mk_acc,
              tot_acc):
    i = pl.program_id(0)
    j = pl.program_id(1)
    ni = pl.num_programs(0)
    nj = pl.num_programs(1)

    @pl.when((i == 0) & (j == 0))
    def _init_tot():
        tot_acc[0] = 0.0

    @pl.when(j == 0)
    def _init():
        r1_acc[...] = jnp.zeros_like(r1_acc)
        r2_acc[...] = jnp.zeros_like(r2_acc)
        mk_acc[...] = jnp.zeros_like(mk_acc)

    inv_tau = 1.0 / _TAU
    s1 = lax.dot_general(u_i[...], v1_j[...], (((1,), (1,)), ((), ())),
                         preferred_element_type=_F32)
    e1 = jnp.exp(s1 * inv_tau)
    s2 = lax.dot_general(v1_i[...], v2_j[...], (((1,), (1,)), ((), ())),
                         preferred_element_type=_F32)
    e2 = jnp.exp(s2 * inv_tau)
    m = (a_ref[...] > 0).astype(_F32)
    r1_acc[...] += jnp.sum(e1, axis=1, keepdims=True)
    r2_acc[...] += jnp.sum(e2, axis=1, keepdims=True)
    mk_acc[...] += jnp.sum((e1 + e2) * m, axis=1, keepdims=True)

    @pl.when(j == nj - 1)
    def _fin():
        masked = mk_acc[...]
        denom = r1_acc[...] - masked + r2_acc[...]
        ct = -jnp.log(masked / denom)
        tot_acc[0] += jnp.sum(ct)

    @pl.when((i == ni - 1) & (j == nj - 1))
    def _emit():
        ct_ref[0] = tot_acc[0]


def _sim(u1, v1, v2, A_aug):
    BM = 512
    BN = 512
    ni, nj = _N // BM, _N // BN
    return pl.pallas_call(
        _sim_body,
        grid=(ni, nj),
        in_specs=[pl.BlockSpec((BM, _HID), lambda i, j: (i, 0)),
                  pl.BlockSpec((BM, _HID), lambda i, j: (i, 0)),
                  pl.BlockSpec((BN, _HID), lambda i, j: (j, 0)),
                  pl.BlockSpec((BN, _HID), lambda i, j: (j, 0)),
                  pl.BlockSpec((BM, BN), lambda i, j: (i, j))],
        out_specs=pl.BlockSpec(memory_space=pltpu.SMEM),
        out_shape=jax.ShapeDtypeStruct((1,), _F32),
        scratch_shapes=[pltpu.VMEM((BM, 1), _F32)] * 3
        + [pltpu.SMEM((1,), _F32)],
    )(u1, v1, v1, v2, A_aug)


def _prop_out_body(a_ref, h_ref, w_ref, b_ref, y_ref):
    h2 = jnp.dot(a_ref[...], h_ref[...], preferred_element_type=_F32)
    y_ref[...] = lax.dot_general(h2, w_ref[...], (((1,), (1,)), ((), ())),
                                 preferred_element_type=_F32) + b_ref[...]


def _prop_out(A, h1, W_cls, b_cls):
    BM = 256
    return pl.pallas_call(
        _prop_out_body,
        grid=(_N // BM,),
        in_specs=[pl.BlockSpec((BM, _N), lambda i: (i, 0)),
                  pl.BlockSpec((_N, _HID), lambda i: (0, 0)),
                  pl.BlockSpec((_NCLS, _HID), lambda i: (0, 0)),
                  pl.BlockSpec((1, _NCLS), lambda i: (0, 0))],
        out_specs=pl.BlockSpec((BM, _NCLS), lambda i: (i, 0)),
        out_shape=jax.ShapeDtypeStruct((_N, _NCLS), _F32),
    )(A, h1, W_cls, b_cls.reshape(1, _NCLS))


_E = 131072
_NTILE = 16
_EPT = _E // _NTILE
_WROWS = 128
_WWORDS = _WROWS * _N
_NWIN = _N // _WROWS
_TWORDS = _WWORDS // _NTILE
_ZWORDS = 16384
_BSTRIDE = _WWORDS + 512
_FENCE = _WWORDS + 16


def _adj_body(edges_hbm, a_ori_hbm, a_aug_hbm,
              er_v, ec_v, flat_v, idx_v, ones_v, zero_v, fence_v, acc_sh, sem):
    cid = lax.axis_index("c")
    sid = lax.axis_index("s")

    def build(g, out_hbm):
        base_e = g * (2 * _E) + sid * _EPT
        pltpu.sync_copy(edges_hbm.at[pl.ds(base_e, _EPT)], er_v)
        pltpu.sync_copy(edges_hbm.at[pl.ds(base_e + _E, _EPT)], ec_v)

        def init_body(i, _):
            s = pl.ds(i * 16, 16)
            flat_v[s] = er_v[s] * _N + ec_v[s]
            return _
        lax.fori_loop(0, _EPT // 16, init_body, None)

        def ones_body(i, _):
            ones_v[pl.ds(i * 16, 16)] = jnp.full((16,), 1.0, _F32)
            return _
        lax.fori_loop(0, _EPT // 16, ones_body, None)

        def zinit_body(i, _):
            zero_v[pl.ds(i * 16, 16)] = jnp.zeros((16,), _F32)
            return _
        lax.fori_loop(0, _ZWORDS // 16, zinit_body, None)

        def zero_region(boff):
            for z in range(_TWORDS // _ZWORDS):
                pltpu.sync_copy(
                    zero_v,
                    acc_sh.at[pl.ds(boff + sid * _TWORDS + z * _ZWORDS,
                                    _ZWORDS)])

        zero_region(0)
        zero_region(_BSTRIDE)
        plsc.subcore_barrier()

        def fence(boff):
            fb = boff + _FENCE + sid * 16
            pltpu.sync_copy(fence_v, acc_sh.at[pl.ds(fb, 16)])
            pltpu.sync_copy(acc_sh.at[pl.ds(fb, 16)], fence_v)

        def emit(w, boff):
            pltpu.sync_copy(
                acc_sh.at[pl.ds(boff + sid * _TWORDS, _TWORDS)],
                out_hbm.at[pl.ds(w * _WWORDS + sid * _TWORDS, _TWORDS)])
            zero_region(boff)

        def win_loop(w, _):
            boff = (w & 1) * _BSTRIDE
            lo = w * _WWORDS

            def win_body(i, _):
                s = pl.ds(i * 16, 16)
                off = flat_v[s] - lo
                ok = (off >= 0) & (off < _WWORDS)
                idx_v[s] = jnp.where(ok, off, _WWORDS) + boff
                return _
            lax.fori_loop(0, _EPT // 16, win_body, None)

            pltpu.sync_copy(ones_v, acc_sh.at[idx_v], add=True)
            fence(boff)
            plsc.subcore_barrier()

            @pl.when(w > 0)
            def _():
                emit(w - 1, (1 - (w & 1)) * _BSTRIDE)
            plsc.subcore_barrier()
            return _
        lax.fori_loop(0, _NWIN, win_loop, None)

        pl.delay(16384)
        emit(_NWIN - 1, ((_NWIN - 1) & 1) * _BSTRIDE)

    @pl.when(cid == 0)
    def _():
        build(0, a_ori_hbm)

    @pl.when(cid == 1)
    def _():
        build(1, a_aug_hbm)


def _build_adjs(edge_index_ori, edge_index_aug):
    edges_flat = jnp.concatenate(
        [edge_index_ori.reshape(-1), edge_index_aug.reshape(-1)])
    mesh = plsc.VectorSubcoreMesh(core_axis_name="c", subcore_axis_name="s")
    f = pl.kernel(
        _adj_body, mesh=mesh,
        out_type=[jax.ShapeDtypeStruct((_N * _N,), _F32)] * 2,
        scratch_types=[
            pltpu.VMEM((_EPT,), jnp.int32),
            pltpu.VMEM((_EPT,), jnp.int32),
            pltpu.VMEM((_EPT,), jnp.int32),
            pltpu.VMEM((_EPT,), jnp.int32),
            pltpu.VMEM((_EPT,), _F32),
            pltpu.VMEM((_ZWORDS,), _F32),
            pltpu.VMEM((16,), _F32),
            pltpu.VMEM_SHARED((2 * _BSTRIDE,), _F32),
            pltpu.SemaphoreType.DMA,
        ],
    )
    a_ori, a_aug = f(edges_flat)
    return a_ori.reshape(_N, _N), a_aug.reshape(_N, _N)


def kernel(x, W_fc, b_fc, W_cls, b_cls, edge_index_ori, edge_index_aug):
    A_ori, A_aug = _build_adjs(edge_index_ori, edge_index_aug)

    h, u1 = _fc(x, W_fc, b_fc)
    p, v1 = _spmv_norm(A_aug, h)
    _, v2 = _spmv_norm(A_aug, p)
    h1 = _spmv(A_ori, h)
    y = _prop_out(A_ori, h1, W_cls, b_cls)

    ct_total = _sim(u1, v1, v2, A_aug)
    loss = (_NLAYER / _N) * ct_total[0]
    return (y, loss)


# speedup vs baseline: 7.6934x; 7.6934x over previous
import functools

import jax
import jax.numpy as jnp
from jax import lax
from jax.experimental import pallas as pl
from jax.experimental.pallas import tpu as pltpu
from jax.experimental.pallas import tpu_sc as plsc

_N, _NF, _HID, _NCLS = 4096, 512, 256, 64
_TAU = 0.5
_NLAYER = 2
_F32 = jnp.float32


def _fc_body(x_ref, w_ref, b_ref, h_ref, u_ref):
    h = lax.dot_general(x_ref[...], w_ref[...], (((1,), (1,)), ((), ())),
                        preferred_element_type=_F32)
    h = h + b_ref[...]
    h_ref[...] = h
    n = jnp.sqrt(jnp.sum(h * h, axis=1, keepdims=True))
    u_ref[...] = h / jnp.maximum(n, 1e-12)


def _fc(x, W, b):
    BM = 512
    return pl.pallas_call(
        _fc_body,
        grid=(_N // BM,),
        in_specs=[pl.BlockSpec((BM, _NF), lambda i: (i, 0)),
                  pl.BlockSpec((_HID, _NF), lambda i: (0, 0)),
                  pl.BlockSpec((1, _HID), lambda i: (0, 0))],
        out_specs=[pl.BlockSpec((BM, _HID), lambda i: (i, 0)),
                   pl.BlockSpec((BM, _HID), lambda i: (i, 0))],
        out_shape=[jax.ShapeDtypeStruct((_N, _HID), _F32)] * 2,
    )(x, W, b.reshape(1, _HID))


def _spmv_norm_body(a_ref, z_ref, p_ref, v_ref):
    p = jnp.dot(a_ref[...], z_ref[...], preferred_element_type=_F32)
    p_ref[...] = p
    n = jnp.sqrt(jnp.sum(p * p, axis=1, keepdims=True))
    v_ref[...] = p / jnp.maximum(n, 1e-12)


def _spmv_norm(A, Z):
    BM = 256
    return pl.pallas_call(
        _spmv_norm_body,
        grid=(_N // BM,),
        in_specs=[pl.BlockSpec((BM, _N), lambda i: (i, 0)),
                  pl.BlockSpec((_N, _HID), lambda i: (0, 0))],
        out_specs=[pl.BlockSpec((BM, _HID), lambda i: (i, 0)),
                   pl.BlockSpec((BM, _HID), lambda i: (i, 0))],
        out_shape=[jax.ShapeDtypeStruct((_N, _HID), _F32)] * 2,
    )(A, Z)


def _spmv_body(a_ref, z_ref, p_ref):
    p_ref[...] = jnp.dot(a_ref[...], z_ref[...], preferred_element_type=_F32)


def _spmv(A, Z):
    BM = 256
    return pl.pallas_call(
        _spmv_body,
        grid=(_N // BM,),
        in_specs=[pl.BlockSpec((BM, _N), lambda i: (i, 0)),
                  pl.BlockSpec((_N, _HID), lambda i: (0, 0))],
        out_specs=pl.BlockSpec((BM, _HID), lambda i: (i, 0)),
        out_shape=jax.ShapeDtypeStruct((_N, _HID), _F32),
    )(A, Z)


def _sim_body(u_i, v1_i, v1_j, v2_j, a_ref, ct_ref, r1_acc, r2_acc, mk_acc,
              tot_acc):
    i = pl.program_id(0)
    j = pl.program_id(1)
    ni = pl.num_programs(0)
    nj = pl.num_programs(1)

    @pl.when((i == 0) & (j == 0))
    def _init_tot():
        tot_acc[0] = 0.0

    @pl.when(j == 0)
    def _init():
        r1_acc[...] = jnp.zeros_like(r1_acc)
        r2_acc[...] = jnp.zeros_like(r2_acc)
        mk_acc[...] = jnp.zeros_like(mk_acc)

    inv_tau = 1.0 / _TAU
    s1 = lax.dot_general(u_i[...], v1_j[...], (((1,), (1,)), ((), ())),
                         preferred_element_type=_F32)
    e1 = jnp.exp(s1 * inv_tau)
    s2 = lax.dot_general(v1_i[...], v2_j[...], (((1,), (1,)), ((), ())),
                         preferred_element_type=_F32)
    e2 = jnp.exp(s2 * inv_tau)
    m = (a_ref[...] > 0).astype(_F32)
    r1_acc[...] += jnp.sum(e1, axis=1, keepdims=True)
    r2_acc[...] += jnp.sum(e2, axis=1, keepdims=True)
    mk_acc[...] += jnp.sum((e1 + e2) * m, axis=1, keepdims=True)

    @pl.when(j == nj - 1)
    def _fin():
        masked = mk_acc[...]
        denom = r1_acc[...] - masked + r2_acc[...]
        ct = -jnp.log(masked / denom)
        tot_acc[0] += jnp.sum(ct)

    @pl.when((i == ni - 1) & (j == nj - 1))
    def _emit():
        ct_ref[0] = tot_acc[0]


def _sim(u1, v1, v2, A_aug):
    BM = 512
    BN = 512
    ni, nj = _N // BM, _N // BN
    return pl.pallas_call(
        _sim_body,
        grid=(ni, nj),
        in_specs=[pl.BlockSpec((BM, _HID), lambda i, j: (i, 0)),
                  pl.BlockSpec((BM, _HID), lambda i, j: (i, 0)),
                  pl.BlockSpec((BN, _HID), lambda i, j: (j, 0)),
                  pl.BlockSpec((BN, _HID), lambda i, j: (j, 0)),
                  pl.BlockSpec((BM, BN), lambda i, j: (i, j))],
        out_specs=pl.BlockSpec(memory_space=pltpu.SMEM),
        out_shape=jax.ShapeDtypeStruct((1,), _F32),
        scratch_shapes=[pltpu.VMEM((BM, 1), _F32)] * 3
        + [pltpu.SMEM((1,), _F32)],
    )(u1, v1, v1, v2, A_aug)


def _prop_out_body(a_ref, h_ref, w_ref, b_ref, y_ref):
    h2 = jnp.dot(a_ref[...], h_ref[...], preferred_element_type=_F32)
    y_ref[...] = lax.dot_general(h2, w_ref[...], (((1,), (1,)), ((), ())),
                                 preferred_element_type=_F32) + b_ref[...]


def _prop_out(A, h1, W_cls, b_cls):
    BM = 256
    return pl.pallas_call(
        _prop_out_body,
        grid=(_N // BM,),
        in_specs=[pl.BlockSpec((BM, _N), lambda i: (i, 0)),
                  pl.BlockSpec((_N, _HID), lambda i: (0, 0)),
                  pl.BlockSpec((_NCLS, _HID), lambda i: (0, 0)),
                  pl.BlockSpec((1, _NCLS), lambda i: (0, 0))],
        out_specs=pl.BlockSpec((BM, _NCLS), lambda i: (i, 0)),
        out_shape=jax.ShapeDtypeStruct((_N, _NCLS), _F32),
    )(A, h1, W_cls, b_cls.reshape(1, _NCLS))


_E = 131072
_NTILE = 16
_EPT = _E // _NTILE
_WROWS = 128
_WWORDS = _WROWS * _N
_NWIN = _N // _WROWS
_TWORDS = _WWORDS // _NTILE
_ZWORDS = 16384
_BSTRIDE = _WWORDS + 512
_FENCE = _WWORDS + 16


def _adj_body(edges_hbm, a_ori_hbm, a_aug_hbm,
              er_v, ec_v, flat_v, idx_v, val_v, zero_v, fence_v, acc_sh, sem):
    cid = lax.axis_index("c")
    sid = lax.axis_index("s")

    def build(g, out_hbm):
        base_e = g * (2 * _E) + sid * _EPT
        pltpu.sync_copy(edges_hbm.at[pl.ds(base_e, _EPT)], er_v)
        pltpu.sync_copy(edges_hbm.at[pl.ds(base_e + _E, _EPT)], ec_v)

        def init_body(i, _):
            s = pl.ds(i * 16, 16)
            flat_v[s] = er_v[s] * _N + ec_v[s]
            return _
        lax.fori_loop(0, _EPT // 16, init_body, None)

        def zinit_body(i, _):
            zero_v[pl.ds(i * 16, 16)] = jnp.zeros((16,), _F32)
            return _
        lax.fori_loop(0, _ZWORDS // 16, zinit_body, None)

        def zero_region(boff):
            for z in range(_TWORDS // _ZWORDS):
                pltpu.sync_copy(
                    zero_v,
                    acc_sh.at[pl.ds(boff + sid * _TWORDS + z * _ZWORDS,
                                    _ZWORDS)])

        zero_region(0)
        zero_region(_BSTRIDE)
        plsc.subcore_barrier()

        def fence(boff):
            fb = boff + _FENCE + sid * 16
            pltpu.sync_copy(fence_v, acc_sh.at[pl.ds(fb, 16)])
            pltpu.sync_copy(acc_sh.at[pl.ds(fb, 16)], fence_v)

        def emit(w, boff):
            pltpu.sync_copy(
                acc_sh.at[pl.ds(boff + sid * _TWORDS, _TWORDS)],
                out_hbm.at[pl.ds(w * _WWORDS + sid * _TWORDS, _TWORDS)])
            zero_region(boff)

        def win_loop(w, _):
            boff = (w & 1) * _BSTRIDE
            lo = w * _WWORDS

            def win_body(i, _):
                s = pl.ds(i * 16, 16)
                off = flat_v[s] - lo
                ok = (off >= 0) & (off < _WWORDS)
                idx_v[s] = (off & (_WWORDS - 1)) + boff
                val_v[s] = jnp.where(ok, 1.0, 0.0).astype(_F32)
                return _
            lax.fori_loop(0, _EPT // 16, win_body, None)

            pltpu.sync_copy(val_v, acc_sh.at[idx_v], add=True)
            fence(boff)
            plsc.subcore_barrier()

            @pl.when(w > 0)
            def _():
                emit(w - 1, (1 - (w & 1)) * _BSTRIDE)
            plsc.subcore_barrier()
            return _
        lax.fori_loop(0, _NWIN, win_loop, None)

        pl.delay(16384)
        emit(_NWIN - 1, ((_NWIN - 1) & 1) * _BSTRIDE)

    @pl.when(cid == 0)
    def _():
        build(0, a_ori_hbm)

    @pl.when(cid == 1)
    def _():
        build(1, a_aug_hbm)


def _build_adjs(edge_index_ori, edge_index_aug):
    edges_flat = jnp.concatenate(
        [edge_index_ori.reshape(-1), edge_index_aug.reshape(-1)])
    mesh = plsc.VectorSubcoreMesh(core_axis_name="c", subcore_axis_name="s")
    f = pl.kernel(
        _adj_body, mesh=mesh,
        out_type=[jax.ShapeDtypeStruct((_N * _N,), _F32)] * 2,
        scratch_types=[
            pltpu.VMEM((_EPT,), jnp.int32),
            pltpu.VMEM((_EPT,), jnp.int32),
            pltpu.VMEM((_EPT,), jnp.int32),
            pltpu.VMEM((_EPT,), jnp.int32),
            pltpu.VMEM((_EPT,), _F32),
            pltpu.VMEM((_ZWORDS,), _F32),
            pltpu.VMEM((16,), _F32),
            pltpu.VMEM_SHARED((2 * _BSTRIDE,), _F32),
            pltpu.SemaphoreType.DMA,
        ],
    )
    a_ori, a_aug = f(edges_flat)
    return a_ori.reshape(_N, _N), a_aug.reshape(_N, _N)


def kernel(x, W_fc, b_fc, W_cls, b_cls, edge_index_ori, edge_index_aug):
    A_ori, A_aug = _build_adjs(edge_index_ori, edge_index_aug)

    h, u1 = _fc(x, W_fc, b_fc)
    p, v1 = _spmv_norm(A_aug, h)
    _, v2 = _spmv_norm(A_aug, p)
    h1 = _spmv(A_ori, h)
    y = _prop_out(A_ori, h1, W_cls, b_cls)

    ct_total = _sim(u1, v1, v2, A_aug)
    loss = (_NLAYER / _N) * ct_total[0]
    return (y, loss)


# speedup vs baseline: 8.2476x; 1.0720x over previous
import functools

import jax
import jax.numpy as jnp
from jax import lax
from jax.experimental import pallas as pl
from jax.experimental.pallas import tpu as pltpu
from jax.experimental.pallas import tpu_sc as plsc

_N, _NF, _HID, _NCLS = 4096, 512, 256, 64
_TAU = 0.5
_NLAYER = 2
_F32 = jnp.float32


def _fc_body(x_ref, w_ref, b_ref, h_ref, u_ref):
    h = lax.dot_general(x_ref[...], w_ref[...], (((1,), (1,)), ((), ())),
                        preferred_element_type=_F32)
    h = h + b_ref[...]
    h_ref[...] = h
    n = jnp.sqrt(jnp.sum(h * h, axis=1, keepdims=True))
    u_ref[...] = h / jnp.maximum(n, 1e-12)


def _fc(x, W, b):
    BM = 512
    return pl.pallas_call(
        _fc_body,
        grid=(_N // BM,),
        in_specs=[pl.BlockSpec((BM, _NF), lambda i: (i, 0)),
                  pl.BlockSpec((_HID, _NF), lambda i: (0, 0)),
                  pl.BlockSpec((1, _HID), lambda i: (0, 0))],
        out_specs=[pl.BlockSpec((BM, _HID), lambda i: (i, 0)),
                   pl.BlockSpec((BM, _HID), lambda i: (i, 0))],
        out_shape=[jax.ShapeDtypeStruct((_N, _HID), _F32)] * 2,
    )(x, W, b.reshape(1, _HID))


def _spmv_norm_body(a_ref, z_ref, p_ref, v_ref):
    p = jnp.dot(a_ref[...], z_ref[...], preferred_element_type=_F32)
    p_ref[...] = p
    n = jnp.sqrt(jnp.sum(p * p, axis=1, keepdims=True))
    v_ref[...] = p / jnp.maximum(n, 1e-12)


def _spmv_norm(A, Z):
    BM = 256
    return pl.pallas_call(
        _spmv_norm_body,
        grid=(_N // BM,),
        in_specs=[pl.BlockSpec((BM, _N), lambda i: (i, 0)),
                  pl.BlockSpec((_N, _HID), lambda i: (0, 0))],
        out_specs=[pl.BlockSpec((BM, _HID), lambda i: (i, 0)),
                   pl.BlockSpec((BM, _HID), lambda i: (i, 0))],
        out_shape=[jax.ShapeDtypeStruct((_N, _HID), _F32)] * 2,
    )(A, Z)


def _spmv_body(a_ref, z_ref, p_ref):
    p_ref[...] = jnp.dot(a_ref[...], z_ref[...], preferred_element_type=_F32)


def _spmv(A, Z):
    BM = 256
    return pl.pallas_call(
        _spmv_body,
        grid=(_N // BM,),
        in_specs=[pl.BlockSpec((BM, _N), lambda i: (i, 0)),
                  pl.BlockSpec((_N, _HID), lambda i: (0, 0))],
        out_specs=pl.BlockSpec((BM, _HID), lambda i: (i, 0)),
        out_shape=jax.ShapeDtypeStruct((_N, _HID), _F32),
    )(A, Z)


def _sim_body(u_i, v1_i, v1_j, v2_j, a_ref, ct_ref, r1_acc, r2_acc, mk_acc,
              tot_acc):
    i = pl.program_id(0)
    j = pl.program_id(1)
    ni = pl.num_programs(0)
    nj = pl.num_programs(1)

    @pl.when((i == 0) & (j == 0))
    def _init_tot():
        tot_acc[0] = 0.0

    @pl.when(j == 0)
    def _init():
        r1_acc[...] = jnp.zeros_like(r1_acc)
        r2_acc[...] = jnp.zeros_like(r2_acc)
        mk_acc[...] = jnp.zeros_like(mk_acc)

    inv_tau = 1.0 / _TAU
    s1 = lax.dot_general(u_i[...], v1_j[...], (((1,), (1,)), ((), ())),
                         preferred_element_type=_F32)
    e1 = jnp.exp(s1 * inv_tau)
    s2 = lax.dot_general(v1_i[...], v2_j[...], (((1,), (1,)), ((), ())),
                         preferred_element_type=_F32)
    e2 = jnp.exp(s2 * inv_tau)
    m = (a_ref[...] > 0).astype(_F32)
    r1_acc[...] += jnp.sum(e1, axis=1, keepdims=True)
    r2_acc[...] += jnp.sum(e2, axis=1, keepdims=True)
    mk_acc[...] += jnp.sum((e1 + e2) * m, axis=1, keepdims=True)

    @pl.when(j == nj - 1)
    def _fin():
        masked = mk_acc[...]
        denom = r1_acc[...] - masked + r2_acc[...]
        ct = -jnp.log(masked / denom)
        tot_acc[0] += jnp.sum(ct)

    @pl.when((i == ni - 1) & (j == nj - 1))
    def _emit():
        ct_ref[0] = tot_acc[0]


def _sim(u1, v1, v2, A_aug):
    BM = 512
    BN = 512
    ni, nj = _N // BM, _N // BN
    return pl.pallas_call(
        _sim_body,
        grid=(ni, nj),
        in_specs=[pl.BlockSpec((BM, _HID), lambda i, j: (i, 0)),
                  pl.BlockSpec((BM, _HID), lambda i, j: (i, 0)),
                  pl.BlockSpec((BN, _HID), lambda i, j: (j, 0)),
                  pl.BlockSpec((BN, _HID), lambda i, j: (j, 0)),
                  pl.BlockSpec((BM, BN), lambda i, j: (i, j))],
        out_specs=pl.BlockSpec(memory_space=pltpu.SMEM),
        out_shape=jax.ShapeDtypeStruct((1,), _F32),
        scratch_shapes=[pltpu.VMEM((BM, 1), _F32)] * 3
        + [pltpu.SMEM((1,), _F32)],
    )(u1, v1, v1, v2, A_aug)


def _prop_out_body(a_ref, h_ref, w_ref, b_ref, y_ref):
    h2 = jnp.dot(a_ref[...], h_ref[...], preferred_element_type=_F32)
    y_ref[...] = lax.dot_general(h2, w_ref[...], (((1,), (1,)), ((), ())),
                                 preferred_element_type=_F32) + b_ref[...]


def _prop_out(A, h1, W_cls, b_cls):
    BM = 256
    return pl.pallas_call(
        _prop_out_body,
        grid=(_N // BM,),
        in_specs=[pl.BlockSpec((BM, _N), lambda i: (i, 0)),
                  pl.BlockSpec((_N, _HID), lambda i: (0, 0)),
                  pl.BlockSpec((_NCLS, _HID), lambda i: (0, 0)),
                  pl.BlockSpec((1, _NCLS), lambda i: (0, 0))],
        out_specs=pl.BlockSpec((BM, _NCLS), lambda i: (i, 0)),
        out_shape=jax.ShapeDtypeStruct((_N, _NCLS), _F32),
    )(A, h1, W_cls, b_cls.reshape(1, _NCLS))


_E = 131072
_NTILE = 16
_EPT = _E // _NTILE
_WROWS = 128
_WWORDS = _WROWS * _N
_NWIN = _N // _WROWS
_TWORDS = _WWORDS // _NTILE
_ZWORDS = 16384
_BSTRIDE = _WWORDS + 512
_FENCE = _WWORDS + 16


def _adj_body(edges_hbm, a_ori_hbm, a_aug_hbm,
              er_v, ec_v, idx_v, val_v, zero_v, fence_v, acc_sh, sem):
    cid = lax.axis_index("c")
    sid = lax.axis_index("s")

    def build(g, out_hbm):
        base_e = g * (2 * _E) + sid * _EPT
        pltpu.sync_copy(edges_hbm.at[pl.ds(base_e, _EPT)], er_v)
        pltpu.sync_copy(edges_hbm.at[pl.ds(base_e + _E, _EPT)], ec_v)

        def init_body(i, _):
            s = pl.ds(i * 16, 16)
            flat = er_v[s] * _N + ec_v[s]
            er_v[s] = flat & (_WWORDS - 1)
            ec_v[s] = lax.shift_right_logical(flat, 19)
            return _
        lax.fori_loop(0, _EPT // 16, init_body, None)

        def zinit_body(i, _):
            zero_v[pl.ds(i * 16, 16)] = jnp.zeros((16,), _F32)
            return _
        lax.fori_loop(0, _ZWORDS // 16, zinit_body, None)

        def zero_region(boff):
            for z in range(_TWORDS // _ZWORDS):
                pltpu.sync_copy(
                    zero_v,
                    acc_sh.at[pl.ds(boff + sid * _TWORDS + z * _ZWORDS,
                                    _ZWORDS)])

        zero_region(0)
        zero_region(_BSTRIDE)
        plsc.subcore_barrier()

        def fence(boff):
            fb = boff + _FENCE + sid * 16
            pltpu.sync_copy(fence_v, acc_sh.at[pl.ds(fb, 16)])
            pltpu.sync_copy(acc_sh.at[pl.ds(fb, 16)], fence_v)

        def emit(w, boff):
            pltpu.sync_copy(
                acc_sh.at[pl.ds(boff + sid * _TWORDS, _TWORDS)],
                out_hbm.at[pl.ds(w * _WWORDS + sid * _TWORDS, _TWORDS)])
            zero_region(boff)

        def win_loop(w, _):
            boff = (w & 1) * _BSTRIDE
            lo = w * _WWORDS

            def win_body(i, _):
                for k in range(4):
                    s = pl.ds(i * 64 + k * 16, 16)
                    val_v[s] = jnp.where(ec_v[s] == w, 1.0, 0.0).astype(_F32)
                    idx_v[s] = er_v[s] + boff
                return _
            lax.fori_loop(0, _EPT // 64, win_body, None)

            pltpu.sync_copy(val_v, acc_sh.at[idx_v], add=True)
            fence(boff)
            plsc.subcore_barrier()

            @pl.when(w > 0)
            def _():
                emit(w - 1, (1 - (w & 1)) * _BSTRIDE)
            plsc.subcore_barrier()
            return _
        lax.fori_loop(0, _NWIN, win_loop, None)

        pl.delay(16384)
        emit(_NWIN - 1, ((_NWIN - 1) & 1) * _BSTRIDE)

    @pl.when(cid == 0)
    def _():
        build(0, a_ori_hbm)

    @pl.when(cid == 1)
    def _():
        build(1, a_aug_hbm)


def _build_adjs(edge_index_ori, edge_index_aug):
    edges_flat = jnp.concatenate(
        [edge_index_ori.reshape(-1), edge_index_aug.reshape(-1)])
    mesh = plsc.VectorSubcoreMesh(core_axis_name="c", subcore_axis_name="s")
    f = pl.kernel(
        _adj_body, mesh=mesh,
        out_type=[jax.ShapeDtypeStruct((_N * _N,), _F32)] * 2,
        scratch_types=[
            pltpu.VMEM((_EPT,), jnp.int32),
            pltpu.VMEM((_EPT,), jnp.int32),
            pltpu.VMEM((_EPT,), jnp.int32),
            pltpu.VMEM((_EPT,), _F32),
            pltpu.VMEM((_ZWORDS,), _F32),
            pltpu.VMEM((16,), _F32),
            pltpu.VMEM_SHARED((2 * _BSTRIDE,), _F32),
            pltpu.SemaphoreType.DMA,
        ],
    )
    a_ori, a_aug = f(edges_flat)
    return a_ori.reshape(_N, _N), a_aug.reshape(_N, _N)


def kernel(x, W_fc, b_fc, W_cls, b_cls, edge_index_ori, edge_index_aug):
    A_ori, A_aug = _build_adjs(edge_index_ori, edge_index_aug)

    h, u1 = _fc(x, W_fc, b_fc)
    p, v1 = _spmv_norm(A_aug, h)
    _, v2 = _spmv_norm(A_aug, p)
    h1 = _spmv(A_ori, h)
    y = _prop_out(A_ori, h1, W_cls, b_cls)

    ct_total = _sim(u1, v1, v2, A_aug)
    loss = (_NLAYER / _N) * ct_total[0]
    return (y, loss)
